# baseline (speedup 1.0000x reference)
import functools

import jax
import jax.numpy as jnp
from jax import lax
from jax.experimental import pallas as pl
from jax.experimental.pallas import tpu as pltpu
from jax.experimental.pallas import tpu_sc as plsc

B, N, M, C1, C2, CO = 4, 8192, 2048, 128, 256, 128
NB = 256
NBM = 512
NB2 = 2048
_F32_MAX = 3.4e38

NSC = 32
PTS = (B * N) // NSC
CH = 32


def _top3_kernel(unknown_ref, known_ref, idx_ref, wts_ref):
    b = pl.program_id(0)

    u = unknown_ref[0]
    k = known_ref[0]

    d = -2.0 * jax.lax.dot_general(u, k, (((1,), (1,)), ((), ())),
                                   preferred_element_type=jnp.float32)
    d = d + jnp.sum(u * u, axis=1, keepdims=True)
    d = d + jnp.sum(k * k, axis=1)[None, :]

    d1 = jnp.min(d, axis=1, keepdims=True)
    dm = jnp.where(d == d1, _F32_MAX, d)
    d2 = jnp.min(dm, axis=1, keepdims=True)
    dm = jnp.where(dm == d2, _F32_MAX, dm)
    d3 = jnp.min(dm, axis=1, keepdims=True)

    colid = jax.lax.broadcasted_iota(jnp.int32, (NB, M), 1)
    big = jnp.int32(M)
    i1 = jnp.min(jnp.where(d == d1, colid, big), axis=1, keepdims=True)
    i2 = jnp.min(jnp.where(d == d2, colid, big), axis=1, keepdims=True)
    i3 = jnp.min(jnp.where(d == d3, colid, big), axis=1, keepdims=True)

    w1 = 1.0 / (d1 + 1e-8)
    w2 = 1.0 / (d2 + 1e-8)
    w3 = 1.0 / (d3 + 1e-8)
    ws = w1 + w2 + w3

    off = b * M
    idx_ref[0] = jnp.concatenate([i1, i2, i3], axis=1) + off
    wts_ref[0] = jnp.concatenate([w1, w2, w3], axis=1) / ws


def _sc_gather_body(idx_hbm, w_hbm, tab_hbm, out_hbm,
                    idxv, wv, rows, acc, sem):
    cid = lax.axis_index("c")
    sid = lax.axis_index("s")
    wid = sid * 2 + cid
    base = wid * PTS

    def chunk(c, carry):
        off = base + c * CH
        pltpu.sync_copy(idx_hbm.at[pl.ds(3 * off, 3 * CH)], idxv)
        pltpu.sync_copy(w_hbm.at[pl.ds(3 * off, 3 * CH)], wv)
        pltpu.async_copy(tab_hbm.at[idxv], rows, sem).wait()

        def point(p, carry2):
            wa = wv[3 * p, :]
            wb = wv[3 * p + 1, :]
            wc = wv[3 * p + 2, :]
            for v in range(C2 // 16):
                sl = pl.ds(v * 16, 16)
                r = (wa * rows[3 * p, sl] + wb * rows[3 * p + 1, sl]
                     + wc * rows[3 * p + 2, sl])
                acc[p, sl] = r
            return carry2

        lax.fori_loop(0, CH, point, 0, unroll=False)
        pltpu.sync_copy(acc, out_hbm.at[pl.ds(off, CH)])
        return carry

    lax.fori_loop(0, PTS // CH, chunk, 0, unroll=False)


def _mlp_kernel(uf_ref, interp_ref, w0_ref, y_ref, stats_ref):
    b = pl.program_id(0)
    nb = pl.program_id(1)

    uf = uf_ref[0]
    it = interp_ref[0]
    w0a = w0_ref[:, :C1]
    w0b = w0_ref[:, C1:]
    y = jnp.dot(w0a, uf, preferred_element_type=jnp.float32)
    y = y + jax.lax.dot_general(w0b, it, (((1,), (1,)), ((), ())),
                                preferred_element_type=jnp.float32)

    y_ref[0] = y

    @pl.when(jnp.logical_and(b == 0, nb == 0))
    def _():
        stats_ref[...] = jnp.zeros_like(stats_ref)

    ps = jnp.sum(y, axis=1, keepdims=True)
    psq = jnp.sum(y * y, axis=1, keepdims=True)
    pad = jnp.zeros((CO, 126), jnp.float32)
    stats_ref[...] += jnp.concatenate([ps, psq, pad], axis=1)


def _bn_kernel(y_ref, stats_ref, params_ref, out_ref):
    cnt = jnp.float32(B * N)
    mean = stats_ref[:, 0:1] / cnt
    ex2 = stats_ref[:, 1:2] / cnt
    var = ex2 - mean * mean
    rstd = jax.lax.rsqrt(var + 1e-5)
    gamma = params_ref[:, 0:1]
    beta = params_ref[:, 1:2]
    y = y_ref[0]
    out = (y - mean) * (rstd * gamma) + beta
    out_ref[0] = jnp.maximum(out, 0.0)


_sc_gather = pl.kernel(
    _sc_gather_body,
    out_type=jax.ShapeDtypeStruct((B * N, C2), jnp.float32),
    mesh=plsc.VectorSubcoreMesh(core_axis_name="c", subcore_axis_name="s"),
    scratch_types=[
        pltpu.VMEM((3 * CH,), jnp.int32),
        pltpu.VMEM((3 * CH, 16), jnp.float32),
        pltpu.VMEM((3 * CH, C2), jnp.float32),
        pltpu.VMEM((CH, C2), jnp.float32),
        pltpu.SemaphoreType.DMA,
    ],
)


@jax.jit
def kernel(unknown, known, unknow_feats, known_feats, W0, gamma0, beta0):
    idx, wts = pl.pallas_call(
        _top3_kernel,
        grid=(B, N // NB),
        in_specs=[
            pl.BlockSpec((1, NB, 3), lambda b, n: (b, n, 0)),
            pl.BlockSpec((1, M, 3), lambda b, n: (b, 0, 0)),
        ],
        out_specs=[
            pl.BlockSpec((1, NB, 3), lambda b, n: (b, n, 0)),
            pl.BlockSpec((1, NB, 3), lambda b, n: (b, n, 0)),
        ],
        out_shape=[
            jax.ShapeDtypeStruct((B, N, 3), jnp.int32),
            jax.ShapeDtypeStruct((B, N, 3), jnp.float32),
        ],
    )(unknown, known)

    table = jnp.transpose(known_feats, (0, 2, 1)).reshape(B * M, C2)
    wts_splat = jnp.broadcast_to(wts.reshape(B * N * 3, 1), (B * N * 3, 16))
    interp = _sc_gather(idx.reshape(B * N * 3), wts_splat, table)
    interp = interp.reshape(B, N, C2)

    y_raw, stats = pl.pallas_call(
        _mlp_kernel,
        grid=(B, N // NBM),
        in_specs=[
            pl.BlockSpec((1, C1, NBM), lambda b, n: (b, 0, n)),
            pl.BlockSpec((1, NBM, C2), lambda b, n: (b, n, 0)),
            pl.BlockSpec((CO, C1 + C2), lambda b, n: (0, 0)),
        ],
        out_specs=[
            pl.BlockSpec((1, CO, NBM), lambda b, n: (b, 0, n)),
            pl.BlockSpec((CO, 128), lambda b, n: (0, 0)),
        ],
        out_shape=[
            jax.ShapeDtypeStruct((B, CO, N), jnp.float32),
            jax.ShapeDtypeStruct((CO, 128), jnp.float32),
        ],
    )(unknow_feats, interp, W0)

    params = jnp.zeros((CO, 128), jnp.float32)
    params = params.at[:, 0].set(gamma0).at[:, 1].set(beta0)

    out = pl.pallas_call(
        _bn_kernel,
        grid=(B, N // NB2),
        in_specs=[
            pl.BlockSpec((1, CO, NB2), lambda b, n: (b, 0, n)),
            pl.BlockSpec((CO, 128), lambda b, n: (0, 0)),
            pl.BlockSpec((CO, 128), lambda b, n: (0, 0)),
        ],
        out_specs=pl.BlockSpec((1, CO, NB2), lambda b, n: (b, 0, n)),
        out_shape=jax.ShapeDtypeStruct((B, CO, N), jnp.float32),
    )(y_raw, stats, params)
    return out


# speedup vs baseline: 1.1804x; 1.1804x over previous
import functools

import jax
import jax.numpy as jnp
from jax import lax
from jax.experimental import pallas as pl
from jax.experimental.pallas import tpu as pltpu
from jax.experimental.pallas import tpu_sc as plsc

B, N, M, C1, C2, CO = 4, 8192, 2048, 128, 256, 128
NB = 256
NBM = 512
NB2 = 2048
_F32_MAX = 3.4e38

NSC = 32
PTS = (B * N) // NSC
CH = 32


def _top3_kernel(unknown_ref, known_ref, idx_ref, wts_ref):
    b = pl.program_id(0)

    u = unknown_ref[0]
    k = known_ref[0]

    d = -2.0 * jax.lax.dot_general(u, k, (((1,), (1,)), ((), ())),
                                   preferred_element_type=jnp.float32)
    d = d + jnp.sum(u * u, axis=1, keepdims=True)
    d = d + jnp.sum(k * k, axis=1)[None, :]

    d1 = jnp.min(d, axis=1, keepdims=True)
    dm = jnp.where(d == d1, _F32_MAX, d)
    d2 = jnp.min(dm, axis=1, keepdims=True)
    dm = jnp.where(dm == d2, _F32_MAX, dm)
    d3 = jnp.min(dm, axis=1, keepdims=True)

    colid = jax.lax.broadcasted_iota(jnp.int32, (NB, M), 1)
    big = jnp.int32(M)
    i1 = jnp.min(jnp.where(d == d1, colid, big), axis=1, keepdims=True)
    i2 = jnp.min(jnp.where(d == d2, colid, big), axis=1, keepdims=True)
    i3 = jnp.min(jnp.where(d == d3, colid, big), axis=1, keepdims=True)

    w1 = 1.0 / (d1 + 1e-8)
    w2 = 1.0 / (d2 + 1e-8)
    w3 = 1.0 / (d3 + 1e-8)
    ws = w1 + w2 + w3

    off = b * M
    idx_ref[0] = jnp.concatenate([i1, i2, i3], axis=1) + off
    wts_ref[0] = jnp.concatenate([w1, w2, w3], axis=1) / ws


def _sc_gather_body(idx_hbm, w_hbm, tab_hbm, out_hbm,
                    idxv, wv0, wv1, rows0, rows1, acc,
                    gsem0, gsem1, wsem0, wsem1):
    cid = lax.axis_index("c")
    sid = lax.axis_index("s")
    wid = sid * 2 + cid
    base = wid * PTS
    nch = PTS // CH

    pltpu.sync_copy(idx_hbm.at[pl.ds(3 * base, 3 * PTS)], idxv)

    def issue(c, rows, wv, gsem, wsem):
        pltpu.async_copy(tab_hbm.at[idxv.at[pl.ds(3 * CH * c, 3 * CH)]],
                         rows, gsem)
        pltpu.async_copy(w_hbm.at[pl.ds(3 * (base + c * CH), 3 * CH)],
                         wv, wsem)

    def run_chunk(c, rows, wv, gsem, wsem, o_rows, o_wv, o_gsem, o_wsem):
        pltpu.make_async_copy(tab_hbm.at[pl.ds(0, 3 * CH)], rows, gsem).wait()
        pltpu.make_async_copy(w_hbm.at[pl.ds(0, 3 * CH)], wv, wsem).wait()

        @pl.when(c + 1 < nch)
        def _():
            issue(c + 1, o_rows, o_wv, o_gsem, o_wsem)

        def point(p, carry2):
            wa = wv[3 * p, :]
            wb = wv[3 * p + 1, :]
            wc = wv[3 * p + 2, :]
            for v in range(C2 // 16):
                sl = pl.ds(v * 16, 16)
                r = (wa * rows[3 * p, sl] + wb * rows[3 * p + 1, sl]
                     + wc * rows[3 * p + 2, sl])
                acc[p, sl] = r
            return carry2

        lax.fori_loop(0, CH, point, 0, unroll=False)
        pltpu.sync_copy(acc, out_hbm.at[pl.ds(base + c * CH, CH)])

    issue(0, rows0, wv0, gsem0, wsem0)

    def chunk(c2, carry):
        c = 2 * c2

        @pl.when(c < nch)
        def _():
            run_chunk(c, rows0, wv0, gsem0, wsem0,
                      rows1, wv1, gsem1, wsem1)

        @pl.when(c + 1 < nch)
        def _():
            run_chunk(c + 1, rows1, wv1, gsem1, wsem1,
                      rows0, wv0, gsem0, wsem0)

        return carry

    lax.fori_loop(0, (nch + 1) // 2, chunk, 0, unroll=False)


def _mlp_kernel(uf_ref, interp_ref, w0_ref, y_ref, stats_ref):
    b = pl.program_id(0)
    nb = pl.program_id(1)

    uf = uf_ref[0]
    it = interp_ref[0]
    w0a = w0_ref[:, :C1]
    w0b = w0_ref[:, C1:]
    y = jnp.dot(w0a, uf, preferred_element_type=jnp.float32)
    y = y + jax.lax.dot_general(w0b, it, (((1,), (1,)), ((), ())),
                                preferred_element_type=jnp.float32)

    y_ref[0] = y

    @pl.when(jnp.logical_and(b == 0, nb == 0))
    def _():
        stats_ref[...] = jnp.zeros_like(stats_ref)

    ps = jnp.sum(y, axis=1, keepdims=True)
    psq = jnp.sum(y * y, axis=1, keepdims=True)
    pad = jnp.zeros((CO, 126), jnp.float32)
    stats_ref[...] += jnp.concatenate([ps, psq, pad], axis=1)


def _bn_kernel(y_ref, stats_ref, params_ref, out_ref):
    cnt = jnp.float32(B * N)
    mean = stats_ref[:, 0:1] / cnt
    ex2 = stats_ref[:, 1:2] / cnt
    var = ex2 - mean * mean
    rstd = jax.lax.rsqrt(var + 1e-5)
    gamma = params_ref[:, 0:1]
    beta = params_ref[:, 1:2]
    y = y_ref[0]
    out = (y - mean) * (rstd * gamma) + beta
    out_ref[0] = jnp.maximum(out, 0.0)


_sc_gather = pl.kernel(
    _sc_gather_body,
    out_type=jax.ShapeDtypeStruct((B * N, C2), jnp.float32),
    mesh=plsc.VectorSubcoreMesh(core_axis_name="c", subcore_axis_name="s"),
    scratch_types=[
        pltpu.VMEM((3 * PTS,), jnp.int32),
        pltpu.VMEM((3 * CH, 16), jnp.float32),
        pltpu.VMEM((3 * CH, 16), jnp.float32),
        pltpu.VMEM((3 * CH, C2), jnp.float32),
        pltpu.VMEM((3 * CH, C2), jnp.float32),
        pltpu.VMEM((CH, C2), jnp.float32),
        pltpu.SemaphoreType.DMA,
        pltpu.SemaphoreType.DMA,
        pltpu.SemaphoreType.DMA,
        pltpu.SemaphoreType.DMA,
    ],
)


@jax.jit
def kernel(unknown, known, unknow_feats, known_feats, W0, gamma0, beta0):
    idx, wts = pl.pallas_call(
        _top3_kernel,
        grid=(B, N // NB),
        in_specs=[
            pl.BlockSpec((1, NB, 3), lambda b, n: (b, n, 0)),
            pl.BlockSpec((1, M, 3), lambda b, n: (b, 0, 0)),
        ],
        out_specs=[
            pl.BlockSpec((1, NB, 3), lambda b, n: (b, n, 0)),
            pl.BlockSpec((1, NB, 3), lambda b, n: (b, n, 0)),
        ],
        out_shape=[
            jax.ShapeDtypeStruct((B, N, 3), jnp.int32),
            jax.ShapeDtypeStruct((B, N, 3), jnp.float32),
        ],
    )(unknown, known)

    table = jnp.transpose(known_feats, (0, 2, 1)).reshape(B * M, C2)
    wts_splat = jnp.broadcast_to(wts.reshape(B * N * 3, 1), (B * N * 3, 16))
    interp = _sc_gather(idx.reshape(B * N * 3), wts_splat, table)
    interp = interp.reshape(B, N, C2)

    y_raw, stats = pl.pallas_call(
        _mlp_kernel,
        grid=(B, N // NBM),
        in_specs=[
            pl.BlockSpec((1, C1, NBM), lambda b, n: (b, 0, n)),
            pl.BlockSpec((1, NBM, C2), lambda b, n: (b, n, 0)),
            pl.BlockSpec((CO, C1 + C2), lambda b, n: (0, 0)),
        ],
        out_specs=[
            pl.BlockSpec((1, CO, NBM), lambda b, n: (b, 0, n)),
            pl.BlockSpec((CO, 128), lambda b, n: (0, 0)),
        ],
        out_shape=[
            jax.ShapeDtypeStruct((B, CO, N), jnp.float32),
            jax.ShapeDtypeStruct((CO, 128), jnp.float32),
        ],
    )(unknow_feats, interp, W0)

    params = jnp.zeros((CO, 128), jnp.float32)
    params = params.at[:, 0].set(gamma0).at[:, 1].set(beta0)

    out = pl.pallas_call(
        _bn_kernel,
        grid=(B, N // NB2),
        in_specs=[
            pl.BlockSpec((1, CO, NB2), lambda b, n: (b, 0, n)),
            pl.BlockSpec((CO, 128), lambda b, n: (0, 0)),
            pl.BlockSpec((CO, 128), lambda b, n: (0, 0)),
        ],
        out_specs=pl.BlockSpec((1, CO, NB2), lambda b, n: (b, 0, n)),
        out_shape=jax.ShapeDtypeStruct((B, CO, N), jnp.float32),
    )(y_raw, stats, params)
    return out


# speedup vs baseline: 1.8625x; 1.5778x over previous
import functools

import jax
import jax.numpy as jnp
from jax.experimental import pallas as pl

B, N, M, C1, C2, CO = 4, 8192, 2048, 128, 256, 128
NB = 256
NB2 = 2048
_F32_MAX = 3.4e38


def _fp_kernel(unknown_ref, ka_ref, uf_ref, kf_ref, w0_ref,
               y_ref, stats_ref):
    b = pl.program_id(0)
    nb = pl.program_id(1)

    ua = unknown_ref[0]
    ka = ka_ref[0]

    d = jax.lax.dot_general(ua, ka, (((1,), (1,)), ((), ())),
                            preferred_element_type=jnp.float32)
    u = ua[:, :3] * -0.5
    su = jnp.sum(u * u, axis=1, keepdims=True)

    d1 = jnp.min(d, axis=1, keepdims=True)
    dm = jnp.where(d == d1, _F32_MAX, d)
    d2 = jnp.min(dm, axis=1, keepdims=True)
    dm = jnp.where(dm == d2, _F32_MAX, dm)
    d3 = jnp.min(dm, axis=1, keepdims=True)

    w1 = 1.0 / ((d1 + su) + 1e-8)
    w2 = 1.0 / ((d2 + su) + 1e-8)
    w3 = 1.0 / ((d3 + su) + 1e-8)
    ws = w1 + w2 + w3
    w1, w2, w3 = w1 / ws, w2 / ws, w3 / ws

    wsp = jnp.where(d == d1, w1,
                    jnp.where(d == d2, w2,
                              jnp.where(d == d3, w3, 0.0)))

    kf = kf_ref[0]
    interp_t = jax.lax.dot_general(kf, wsp, (((1,), (1,)), ((), ())),
                                   preferred_element_type=jnp.float32)

    uf = uf_ref[0]
    w0a = w0_ref[:, :C1]
    w0b = w0_ref[:, C1:]
    y = jnp.dot(w0a, uf, preferred_element_type=jnp.float32)
    y = y + jnp.dot(w0b, interp_t, preferred_element_type=jnp.float32)

    y_ref[0] = y

    @pl.when(jnp.logical_and(b == 0, nb == 0))
    def _():
        stats_ref[...] = jnp.zeros_like(stats_ref)

    ps = jnp.sum(y, axis=1, keepdims=True)
    psq = jnp.sum(y * y, axis=1, keepdims=True)
    pad = jnp.zeros((CO, 126), jnp.float32)
    stats_ref[...] += jnp.concatenate([ps, psq, pad], axis=1)


def _bn_kernel(y_ref, stats_ref, params_ref, out_ref):
    cnt = jnp.float32(B * N)
    mean = stats_ref[:, 0:1] / cnt
    ex2 = stats_ref[:, 1:2] / cnt
    var = ex2 - mean * mean
    rstd = jax.lax.rsqrt(var + 1e-5)
    gamma = params_ref[:, 0:1]
    beta = params_ref[:, 1:2]
    y = y_ref[0]
    out = (y - mean) * (rstd * gamma) + beta
    out_ref[0] = jnp.maximum(out, 0.0)


@jax.jit
def kernel(unknown, known, unknow_feats, known_feats, W0, gamma0, beta0):
    n_blocks = N // NB
    known_aug = jnp.concatenate(
        [known, jnp.sum(known * known, axis=2, keepdims=True)], axis=2)
    unknown_aug = jnp.concatenate(
        [-2.0 * unknown, jnp.ones((B, N, 1), jnp.float32)], axis=2)
    y_raw, stats = pl.pallas_call(
        _fp_kernel,
        grid=(B, n_blocks),
        in_specs=[
            pl.BlockSpec((1, NB, 4), lambda b, n: (b, n, 0)),
            pl.BlockSpec((1, M, 4), lambda b, n: (b, 0, 0)),
            pl.BlockSpec((1, C1, NB), lambda b, n: (b, 0, n)),
            pl.BlockSpec((1, C2, M), lambda b, n: (b, 0, 0)),
            pl.BlockSpec((CO, C1 + C2), lambda b, n: (0, 0)),
        ],
        out_specs=[
            pl.BlockSpec((1, CO, NB), lambda b, n: (b, 0, n)),
            pl.BlockSpec((CO, 128), lambda b, n: (0, 0)),
        ],
        out_shape=[
            jax.ShapeDtypeStruct((B, CO, N), jnp.float32),
            jax.ShapeDtypeStruct((CO, 128), jnp.float32),
        ],
    )(unknown_aug, known_aug, unknow_feats, known_feats, W0)

    params = jnp.zeros((CO, 128), jnp.float32)
    params = params.at[:, 0].set(gamma0).at[:, 1].set(beta0)

    out = pl.pallas_call(
        _bn_kernel,
        grid=(B, N // NB2),
        in_specs=[
            pl.BlockSpec((1, CO, NB2), lambda b, n: (b, 0, n)),
            pl.BlockSpec((CO, 128), lambda b, n: (0, 0)),
            pl.BlockSpec((CO, 128), lambda b, n: (0, 0)),
        ],
        out_specs=pl.BlockSpec((1, CO, NB2), lambda b, n: (b, 0, n)),
        out_shape=jax.ShapeDtypeStruct((B, CO, N), jnp.float32),
    )(y_raw, stats, params)
    return out


# speedup vs baseline: 2.4883x; 1.3360x over previous
import functools

import jax
import jax.numpy as jnp
from jax.experimental import pallas as pl

B, N, M, C1, C2, CO = 4, 8192, 2048, 128, 256, 128
NB = 512
NB2 = 2048
_F32_MAX = 3.4e38


def _fp_kernel(unknown_ref, ka_ref, uf_ref, kf_ref, w0_ref,
               y_ref, stats_ref):
    b = pl.program_id(0)
    nb = pl.program_id(1)

    u = unknown_ref[0]
    k = ka_ref[0]

    d = -2.0 * jax.lax.dot_general(u, k, (((1,), (1,)), ((), ())),
                                   preferred_element_type=jnp.float32)
    d = d + jnp.sum(u * u, axis=1, keepdims=True)
    d = d + jnp.sum(k * k, axis=1)[None, :]

    d1 = jnp.min(d, axis=1, keepdims=True)
    dm = jnp.where(d == d1, _F32_MAX, d)
    d2 = jnp.min(dm, axis=1, keepdims=True)
    dm = jnp.where(dm == d2, _F32_MAX, dm)
    d3 = jnp.min(dm, axis=1, keepdims=True)

    w1 = 1.0 / (d1 + 1e-8)
    w2 = 1.0 / (d2 + 1e-8)
    w3 = 1.0 / (d3 + 1e-8)
    ws = w1 + w2 + w3
    w1, w2, w3 = w1 / ws, w2 / ws, w3 / ws

    wsp = jnp.where(d == d1, w1,
                    jnp.where(d == d2, w2,
                              jnp.where(d == d3, w3, 0.0)))

    kf = kf_ref[0]
    interp_t = jax.lax.dot_general(kf, wsp, (((1,), (1,)), ((), ())),
                                   preferred_element_type=jnp.float32)

    uf = uf_ref[0]
    w0a = w0_ref[:, :C1]
    w0b = w0_ref[:, C1:]
    y = jnp.dot(w0a, uf, preferred_element_type=jnp.float32)
    y = y + jnp.dot(w0b, interp_t, preferred_element_type=jnp.float32)

    y_ref[0] = y

    @pl.when(jnp.logical_and(b == 0, nb == 0))
    def _():
        stats_ref[...] = jnp.zeros_like(stats_ref)

    ps = jnp.sum(y, axis=1, keepdims=True)
    psq = jnp.sum(y * y, axis=1, keepdims=True)
    pad = jnp.zeros((CO, 126), jnp.float32)
    stats_ref[...] += jnp.concatenate([ps, psq, pad], axis=1)


def _bn_kernel(y_ref, stats_ref, params_ref, out_ref):
    cnt = jnp.float32(B * N)
    mean = stats_ref[:, 0:1] / cnt
    ex2 = stats_ref[:, 1:2] / cnt
    var = ex2 - mean * mean
    rstd = jax.lax.rsqrt(var + 1e-5)
    gamma = params_ref[:, 0:1]
    beta = params_ref[:, 1:2]
    y = y_ref[0]
    out = (y - mean) * (rstd * gamma) + beta
    out_ref[0] = jnp.maximum(out, 0.0)


@jax.jit
def kernel(unknown, known, unknow_feats, known_feats, W0, gamma0, beta0):
    n_blocks = N // NB
    y_raw, stats = pl.pallas_call(
        _fp_kernel,
        grid=(B, n_blocks),
        in_specs=[
            pl.BlockSpec((1, NB, 3), lambda b, n: (b, n, 0)),
            pl.BlockSpec((1, M, 3), lambda b, n: (b, 0, 0)),
            pl.BlockSpec((1, C1, NB), lambda b, n: (b, 0, n)),
            pl.BlockSpec((1, C2, M), lambda b, n: (b, 0, 0)),
            pl.BlockSpec((CO, C1 + C2), lambda b, n: (0, 0)),
        ],
        out_specs=[
            pl.BlockSpec((1, CO, NB), lambda b, n: (b, 0, n)),
            pl.BlockSpec((CO, 128), lambda b, n: (0, 0)),
        ],
        out_shape=[
            jax.ShapeDtypeStruct((B, CO, N), jnp.float32),
            jax.ShapeDtypeStruct((CO, 128), jnp.float32),
        ],
    )(unknown, known, unknow_feats, known_feats, W0)

    params = jnp.zeros((CO, 128), jnp.float32)
    params = params.at[:, 0].set(gamma0).at[:, 1].set(beta0)

    out = pl.pallas_call(
        _bn_kernel,
        grid=(B, N // NB2),
        in_specs=[
            pl.BlockSpec((1, CO, NB2), lambda b, n: (b, 0, n)),
            pl.BlockSpec((CO, 128), lambda b, n: (0, 0)),
            pl.BlockSpec((CO, 128), lambda b, n: (0, 0)),
        ],
        out_specs=pl.BlockSpec((1, CO, NB2), lambda b, n: (b, 0, n)),
        out_shape=jax.ShapeDtypeStruct((B, CO, N), jnp.float32),
    )(y_raw, stats, params)
    return out


# speedup vs baseline: 2.6971x; 1.0839x over previous
import functools

import jax
import jax.numpy as jnp
from jax.experimental import pallas as pl

B, N, M, C1, C2, CO = 4, 8192, 2048, 128, 256, 128
NB = 1024
NB2 = 2048
_F32_MAX = 3.4e38


def _fp_kernel(unknown_ref, ka_ref, uf_ref, kf_ref, w0_ref,
               y_ref, stats_ref):
    b = pl.program_id(0)
    nb = pl.program_id(1)

    u = unknown_ref[0]
    k = ka_ref[0]

    d = -2.0 * jax.lax.dot_general(u, k, (((1,), (1,)), ((), ())),
                                   preferred_element_type=jnp.float32)
    d = d + jnp.sum(u * u, axis=1, keepdims=True)
    d = d + jnp.sum(k * k, axis=1)[None, :]

    d1 = jnp.min(d, axis=1, keepdims=True)
    dm = jnp.where(d == d1, _F32_MAX, d)
    d2 = jnp.min(dm, axis=1, keepdims=True)
    dm = jnp.where(dm == d2, _F32_MAX, dm)
    d3 = jnp.min(dm, axis=1, keepdims=True)

    w1 = 1.0 / (d1 + 1e-8)
    w2 = 1.0 / (d2 + 1e-8)
    w3 = 1.0 / (d3 + 1e-8)
    ws = w1 + w2 + w3
    w1, w2, w3 = w1 / ws, w2 / ws, w3 / ws

    wsp = jnp.where(d == d1, w1,
                    jnp.where(d == d2, w2,
                              jnp.where(d == d3, w3, 0.0)))

    kf = kf_ref[0]
    interp_t = jax.lax.dot_general(kf, wsp, (((1,), (1,)), ((), ())),
                                   preferred_element_type=jnp.float32)

    uf = uf_ref[0]
    w0a = w0_ref[:, :C1]
    w0b = w0_ref[:, C1:]
    y = jnp.dot(w0a, uf, preferred_element_type=jnp.float32)
    y = y + jnp.dot(w0b, interp_t, preferred_element_type=jnp.float32)

    y_ref[0] = y

    @pl.when(jnp.logical_and(b == 0, nb == 0))
    def _():
        stats_ref[...] = jnp.zeros_like(stats_ref)

    ps = jnp.sum(y, axis=1, keepdims=True)
    psq = jnp.sum(y * y, axis=1, keepdims=True)
    pad = jnp.zeros((CO, 126), jnp.float32)
    stats_ref[...] += jnp.concatenate([ps, psq, pad], axis=1)


def _bn_kernel(y_ref, stats_ref, params_ref, out_ref):
    cnt = jnp.float32(B * N)
    mean = stats_ref[:, 0:1] / cnt
    ex2 = stats_ref[:, 1:2] / cnt
    var = ex2 - mean * mean
    rstd = jax.lax.rsqrt(var + 1e-5)
    gamma = params_ref[:, 0:1]
    beta = params_ref[:, 1:2]
    y = y_ref[0]
    out = (y - mean) * (rstd * gamma) + beta
    out_ref[0] = jnp.maximum(out, 0.0)


@jax.jit
def kernel(unknown, known, unknow_feats, known_feats, W0, gamma0, beta0):
    n_blocks = N // NB
    y_raw, stats = pl.pallas_call(
        _fp_kernel,
        grid=(B, n_blocks),
        in_specs=[
            pl.BlockSpec((1, NB, 3), lambda b, n: (b, n, 0)),
            pl.BlockSpec((1, M, 3), lambda b, n: (b, 0, 0)),
            pl.BlockSpec((1, C1, NB), lambda b, n: (b, 0, n)),
            pl.BlockSpec((1, C2, M), lambda b, n: (b, 0, 0)),
            pl.BlockSpec((CO, C1 + C2), lambda b, n: (0, 0)),
        ],
        out_specs=[
            pl.BlockSpec((1, CO, NB), lambda b, n: (b, 0, n)),
            pl.BlockSpec((CO, 128), lambda b, n: (0, 0)),
        ],
        out_shape=[
            jax.ShapeDtypeStruct((B, CO, N), jnp.float32),
            jax.ShapeDtypeStruct((CO, 128), jnp.float32),
        ],
    )(unknown, known, unknow_feats, known_feats, W0)

    params = jnp.zeros((CO, 128), jnp.float32)
    params = params.at[:, 0].set(gamma0).at[:, 1].set(beta0)

    out = pl.pallas_call(
        _bn_kernel,
        grid=(B, N // NB2),
        in_specs=[
            pl.BlockSpec((1, CO, NB2), lambda b, n: (b, 0, n)),
            pl.BlockSpec((CO, 128), lambda b, n: (0, 0)),
            pl.BlockSpec((CO, 128), lambda b, n: (0, 0)),
        ],
        out_specs=pl.BlockSpec((1, CO, NB2), lambda b, n: (b, 0, n)),
        out_shape=jax.ShapeDtypeStruct((B, CO, N), jnp.float32),
    )(y_raw, stats, params)
    return out
